# Initial kernel scaffold; baseline (speedup 1.0000x reference)
import functools

import jax
import jax.numpy as jnp
from jax import lax
from jax.experimental import pallas as pl
from jax.experimental.pallas import tpu as pltpu
from jax.experimental.pallas import tpu_sc as plsc

N = 10000
E = 320000
D_IN = 128
D_HID = 128
D_OUT = 64

NC = 2
NS = 16
NW = NC * NS
EPW = E // NW
K = 80
NCHUNK = EPW // K
RPT = N // NS
DEG_W = 16

_mesh = plsc.VectorSubcoreMesh(core_axis_name="c", subcore_axis_name="s")


def _worker(c, s):
    return s * NC + c


@functools.partial(
    pl.kernel,
    out_type=jax.ShapeDtypeStruct((NC, N, DEG_W), jnp.float32),
    mesh=_mesh,
    scratch_types=[
        pltpu.VMEM((NCHUNK, K), jnp.int32),
        pltpu.VMEM((K, DEG_W), jnp.float32),
        pltpu.VMEM_SHARED((N, DEG_W), jnp.float32),
    ],
)
def _deg_kernel(dst_hbm, ones_hbm, zeros_hbm, out_hbm, didx, ones_v, acc):
    c = lax.axis_index("c")
    s = lax.axis_index("s")
    wid = _worker(c, s)
    pltpu.sync_copy(ones_hbm, ones_v)
    pltpu.sync_copy(dst_hbm.at[pl.ds(wid * EPW, EPW)], didx)
    pltpu.sync_copy(zeros_hbm.at[pl.ds(s * RPT, RPT)],
                    acc.at[pl.ds(s * RPT, RPT)])
    plsc.subcore_barrier()

    def chunk(i, carry):
        pltpu.sync_copy(ones_v, acc.at[didx.at[i]], add=True)
        return carry

    lax.fori_loop(0, NCHUNK, chunk, 0)
    plsc.subcore_barrier()
    pltpu.sync_copy(acc.at[pl.ds(s * RPT, RPT)],
                    out_hbm.at[c, pl.ds(s * RPT, RPT)])


def _make_agg(D):
    @functools.partial(
        pl.kernel,
        out_type=jax.ShapeDtypeStruct((NC, N, D), jnp.float32),
        mesh=_mesh,
        scratch_types=[
            pltpu.VMEM((NCHUNK, K), jnp.int32),
            pltpu.VMEM((NCHUNK, K), jnp.int32),
            pltpu.VMEM((K, D), jnp.float32),
            pltpu.VMEM_SHARED((N, D), jnp.float32),
            pltpu.SemaphoreType.DMA,
        ],
    )
    def agg(g_hbm, src_hbm, dst_hbm, zeros_hbm, out_hbm,
            sidx, didx, rows, acc, sem):
        c = lax.axis_index("c")
        s = lax.axis_index("s")
        wid = _worker(c, s)
        pltpu.sync_copy(src_hbm.at[pl.ds(wid * EPW, EPW)], sidx)
        pltpu.sync_copy(dst_hbm.at[pl.ds(wid * EPW, EPW)], didx)
        pltpu.sync_copy(zeros_hbm.at[pl.ds(s * RPT, RPT)],
                        acc.at[pl.ds(s * RPT, RPT)])
        plsc.subcore_barrier()

        def chunk(i, carry):
            pltpu.async_copy(g_hbm.at[sidx.at[i]], rows, sem).wait()
            pltpu.sync_copy(rows, acc.at[didx.at[i]], add=True)
            return carry

        lax.fori_loop(0, NCHUNK, chunk, 0)
        plsc.subcore_barrier()
        pltpu.sync_copy(acc.at[pl.ds(s * RPT, RPT)],
                        out_hbm.at[c, pl.ds(s * RPT, RPT)])

    return agg


_agg_hid = _make_agg(D_HID)
_agg_out = _make_agg(D_OUT)


_BLK = 500


def _dinv(d0_ref, d1_ref):
    deg = d0_ref[:, 0:1] + d1_ref[:, 0:1] + 1.0
    return lax.rsqrt(deg)


def _g1_body(d0_ref, d1_ref, x_ref, w_ref, o_ref):
    h = jnp.dot(x_ref[...], w_ref[...], preferred_element_type=jnp.float32)
    o_ref[...] = h * _dinv(d0_ref, d1_ref)


def _mid_body(d0_ref, d1_ref, a0_ref, a1_ref, g1_ref, b1_ref, w2_ref, o_ref):
    dinv = _dinv(d0_ref, d1_ref)
    pre = (a0_ref[...] + a1_ref[...] + g1_ref[...]) * dinv + b1_ref[...]
    act = jnp.maximum(pre, 0.0)
    h = jnp.dot(act, w2_ref[...], preferred_element_type=jnp.float32)
    o_ref[...] = h * dinv


def _out_body(d0_ref, d1_ref, a0_ref, a1_ref, g2_ref, b2_ref, o_ref):
    dinv = _dinv(d0_ref, d1_ref)
    o_ref[...] = (a0_ref[...] + a1_ref[...] + g2_ref[...]) * dinv + b2_ref[...]


def _row_spec(d):
    return pl.BlockSpec((_BLK, d), lambda i: (i, 0))


def _full_spec(r, d):
    return pl.BlockSpec((r, d), lambda i: (0, 0))


_g1_call = pl.pallas_call(
    _g1_body,
    grid=(N // _BLK,),
    in_specs=[_row_spec(DEG_W), _row_spec(DEG_W), _row_spec(D_IN),
              _full_spec(D_IN, D_HID)],
    out_specs=_row_spec(D_HID),
    out_shape=jax.ShapeDtypeStruct((N, D_HID), jnp.float32),
)

_mid_call = pl.pallas_call(
    _mid_body,
    grid=(N // _BLK,),
    in_specs=[_row_spec(DEG_W), _row_spec(DEG_W), _row_spec(D_HID),
              _row_spec(D_HID), _row_spec(D_HID), _full_spec(1, D_HID),
              _full_spec(D_HID, D_OUT)],
    out_specs=_row_spec(D_OUT),
    out_shape=jax.ShapeDtypeStruct((N, D_OUT), jnp.float32),
)

_out_call = pl.pallas_call(
    _out_body,
    grid=(N // _BLK,),
    in_specs=[_row_spec(DEG_W), _row_spec(DEG_W), _row_spec(D_OUT),
              _row_spec(D_OUT), _row_spec(D_OUT), _full_spec(1, D_OUT)],
    out_specs=_row_spec(D_OUT),
    out_shape=jax.ShapeDtypeStruct((N, D_OUT), jnp.float32),
)


@jax.jit
def kernel(x, edge_index, W1, b1, W2, b2):
    src = edge_index[0]
    dst = edge_index[1]
    ones_deg = jnp.ones((K, DEG_W), jnp.float32)
    zeros_deg = jnp.zeros((N, DEG_W), jnp.float32)
    zeros_hid = jnp.zeros((N, D_HID), jnp.float32)
    zeros_out = jnp.zeros((N, D_OUT), jnp.float32)

    degacc = _deg_kernel(dst, ones_deg, zeros_deg)
    d0, d1 = degacc[0], degacc[1]

    g1 = _g1_call(d0, d1, x, W1)
    agg1 = _agg_hid(g1, src, dst, zeros_hid)
    g2 = _mid_call(d0, d1, agg1[0], agg1[1], g1,
                   b1.reshape(1, D_HID), W2)
    agg2 = _agg_out(g2, src, dst, zeros_out)
    out = _out_call(d0, d1, agg2[0], agg2[1], g2,
                    b2.reshape(1, D_OUT))
    return out


# speedup vs baseline: 16.8407x; 16.8407x over previous
import functools

import jax
import jax.numpy as jnp
from jax import lax
from jax.experimental import pallas as pl
from jax.experimental.pallas import tpu as pltpu
from jax.experimental.pallas import tpu_sc as plsc

N = 10000
E = 320000
D_IN = 128
D_HID = 128
D_OUT = 64

NC = 2
NS = 16
NW = NC * NS
EPW = E // NW
K = 80
NCHUNK = EPW // K
NP = 10240
RPT = NP // NS
DEG_W = 128

_mesh = plsc.VectorSubcoreMesh(core_axis_name="c", subcore_axis_name="s")


def _worker(c, s):
    return s * NC + c


@functools.partial(
    pl.kernel,
    out_type=jax.ShapeDtypeStruct((NC, NP, DEG_W), jnp.float32),
    mesh=_mesh,
    scratch_types=[
        pltpu.VMEM((NCHUNK, K), jnp.int32),
        pltpu.VMEM((K, DEG_W), jnp.float32),
        pltpu.VMEM_SHARED((NP, DEG_W), jnp.float32),
    ],
)
def _deg_kernel(dst_hbm, ones_hbm, zeros_hbm, out_hbm, didx, ones_v, acc):
    c = lax.axis_index("c")
    s = lax.axis_index("s")
    wid = _worker(c, s)
    pltpu.sync_copy(ones_hbm, ones_v)
    pltpu.sync_copy(dst_hbm.at[wid], didx)
    pltpu.sync_copy(zeros_hbm.at[pl.ds(s * RPT, RPT)],
                    acc.at[pl.ds(s * RPT, RPT)])
    plsc.subcore_barrier()

    def chunk(i, carry):
        pltpu.sync_copy(ones_v, acc.at[didx.at[i]], add=True)
        return carry

    lax.fori_loop(0, NCHUNK, chunk, 0)
    plsc.subcore_barrier()
    pltpu.sync_copy(acc.at[pl.ds(s * RPT, RPT)],
                    out_hbm.at[c, pl.ds(s * RPT, RPT)])


def _make_agg(D):
    @functools.partial(
        pl.kernel,
        out_type=jax.ShapeDtypeStruct((NC, NP, D), jnp.float32),
        mesh=_mesh,
        scratch_types=[
            pltpu.VMEM((NCHUNK, K), jnp.int32),
            pltpu.VMEM((NCHUNK, K), jnp.int32),
            pltpu.VMEM((K, D), jnp.float32),
            pltpu.VMEM_SHARED((NP, D), jnp.float32),
            pltpu.SemaphoreType.DMA,
        ],
    )
    def agg(g_hbm, src_hbm, dst_hbm, zeros_hbm, out_hbm,
            sidx, didx, rows, acc, sem):
        c = lax.axis_index("c")
        s = lax.axis_index("s")
        wid = _worker(c, s)
        pltpu.sync_copy(src_hbm.at[wid], sidx)
        pltpu.sync_copy(dst_hbm.at[wid], didx)
        pltpu.sync_copy(zeros_hbm.at[pl.ds(s * RPT, RPT)],
                        acc.at[pl.ds(s * RPT, RPT)])
        plsc.subcore_barrier()

        def chunk(i, carry):
            pltpu.async_copy(g_hbm.at[sidx.at[i]], rows, sem).wait()
            pltpu.sync_copy(rows, acc.at[didx.at[i]], add=True)
            return carry

        lax.fori_loop(0, NCHUNK, chunk, 0)
        plsc.subcore_barrier()
        pltpu.sync_copy(acc.at[pl.ds(s * RPT, RPT)],
                        out_hbm.at[c, pl.ds(s * RPT, RPT)])

    return agg


_agg_hid = _make_agg(D_HID)


_BLK = 1000


def _dinv(d0_ref, d1_ref):
    deg = d0_ref[:, 0:1] + d1_ref[:, 0:1] + 1.0
    return lax.rsqrt(deg)


def _g1_body(d0_ref, d1_ref, x_ref, w_ref, o_ref):
    h = jnp.dot(x_ref[...], w_ref[...], preferred_element_type=jnp.float32)
    o_ref[...] = h * _dinv(d0_ref, d1_ref)


def _mid_body(d0_ref, d1_ref, a0_ref, a1_ref, g1_ref, b1_ref, o_ref):
    dinv = _dinv(d0_ref, d1_ref)
    pre = (a0_ref[...] + a1_ref[...] + g1_ref[...]) * dinv + b1_ref[...]
    o_ref[...] = jnp.maximum(pre, 0.0) * dinv


def _out_body(d0_ref, d1_ref, a0_ref, a1_ref, s1_ref, w2_ref, b2_ref, o_ref):
    dinv = _dinv(d0_ref, d1_ref)
    pre = (a0_ref[...] + a1_ref[...] + s1_ref[...]) * dinv
    h = jnp.dot(pre, w2_ref[...], preferred_element_type=jnp.float32)
    o_ref[...] = h + b2_ref[...]


def _row_spec(d):
    return pl.BlockSpec((_BLK, d), lambda i: (i, 0))


def _full_spec(r, d):
    return pl.BlockSpec((r, d), lambda i: (0, 0))


_g1_call = pl.pallas_call(
    _g1_body,
    grid=(N // _BLK,),
    in_specs=[_row_spec(DEG_W), _row_spec(DEG_W), _row_spec(D_IN),
              _full_spec(D_IN, D_HID)],
    out_specs=_row_spec(D_HID),
    out_shape=jax.ShapeDtypeStruct((N, D_HID), jnp.float32),
)

_mid_call = pl.pallas_call(
    _mid_body,
    grid=(N // _BLK,),
    in_specs=[_row_spec(DEG_W), _row_spec(DEG_W), _row_spec(D_HID),
              _row_spec(D_HID), _row_spec(D_HID), _full_spec(1, D_HID)],
    out_specs=_row_spec(D_HID),
    out_shape=jax.ShapeDtypeStruct((N, D_HID), jnp.float32),
)

_out_call = pl.pallas_call(
    _out_body,
    grid=(N // _BLK,),
    in_specs=[_row_spec(DEG_W), _row_spec(DEG_W), _row_spec(D_HID),
              _row_spec(D_HID), _row_spec(D_HID), _full_spec(D_HID, D_OUT),
              _full_spec(1, D_OUT)],
    out_specs=_row_spec(D_OUT),
    out_shape=jax.ShapeDtypeStruct((N, D_OUT), jnp.float32),
)


@jax.jit
def kernel(x, edge_index, W1, b1, W2, b2):
    src = edge_index[0].reshape(NW, NCHUNK, K)
    dst = edge_index[1].reshape(NW, NCHUNK, K)
    ones_deg = jnp.ones((K, DEG_W), jnp.float32)
    zeros_deg = jnp.zeros((NP, DEG_W), jnp.float32)
    zeros_hid = jnp.zeros((NP, D_HID), jnp.float32)

    degacc = _deg_kernel(dst, ones_deg, zeros_deg)
    d0, d1 = degacc[0], degacc[1]

    g1 = _g1_call(d0, d1, x, W1)
    agg1 = _agg_hid(g1, src, dst, zeros_hid)
    s1 = _mid_call(d0, d1, agg1[0], agg1[1], g1,
                   b1.reshape(1, D_HID))
    agg2 = _agg_hid(s1, src, dst, zeros_hid)
    out = _out_call(d0, d1, agg2[0], agg2[1], s1, W2,
                    b2.reshape(1, D_OUT))
    return out


# speedup vs baseline: 18.8858x; 1.1214x over previous
import functools

import jax
import jax.numpy as jnp
from jax import lax
from jax.experimental import pallas as pl
from jax.experimental.pallas import tpu as pltpu
from jax.experimental.pallas import tpu_sc as plsc

N = 10000
E = 320000
D_IN = 128
D_HID = 128
D_OUT = 64

NC = 2
NS = 16
NW = NC * NS
EPW = E // NW
K = 128
NCHUNK = EPW // K
KT = EPW - NCHUNK * K
NP = 10112
RPT = NP // NS
DEG_W = 128

_mesh = plsc.VectorSubcoreMesh(core_axis_name="c", subcore_axis_name="s")


def _worker(c, s):
    return s * NC + c


@functools.partial(
    pl.kernel,
    out_type=jax.ShapeDtypeStruct((NC, NP, DEG_W), jnp.float32),
    mesh=_mesh,
    scratch_types=[
        pltpu.VMEM((NCHUNK, K), jnp.int32),
        pltpu.VMEM((1, KT), jnp.int32),
        pltpu.VMEM((K, DEG_W), jnp.float32),
        pltpu.VMEM_SHARED((NP, DEG_W), jnp.float32),
        pltpu.SemaphoreType.DMA,
    ],
)
def _deg_kernel(dst_hbm, dstt_hbm, ones_hbm, zeros_hbm, out_hbm,
                didx, didxt, ones_v, acc, sem):
    c = lax.axis_index("c")
    s = lax.axis_index("s")
    wid = _worker(c, s)
    pltpu.sync_copy(ones_hbm, ones_v)
    pltpu.sync_copy(dst_hbm.at[wid], didx)
    pltpu.sync_copy(dstt_hbm.at[wid], didxt)
    pltpu.sync_copy(zeros_hbm.at[pl.ds(s * RPT, RPT)],
                    acc.at[pl.ds(s * RPT, RPT)])
    plsc.subcore_barrier()

    def chunk(i, carry):
        pltpu.async_copy(ones_v, acc.at[didx.at[i]], sem, add=True)
        return carry

    lax.fori_loop(0, NCHUNK, chunk, 0)
    pltpu.async_copy(ones_v.at[pl.ds(0, KT)], acc.at[didxt.at[0]], sem,
                     add=True)

    def drain(i, carry):
        pltpu.make_async_copy(ones_v, acc.at[didx.at[i]], sem).wait()
        return carry

    lax.fori_loop(0, NCHUNK, drain, 0)
    pltpu.make_async_copy(ones_v.at[pl.ds(0, KT)], acc.at[didxt.at[0]],
                          sem).wait()
    plsc.subcore_barrier()
    pltpu.sync_copy(acc.at[pl.ds(s * RPT, RPT)],
                    out_hbm.at[c, pl.ds(s * RPT, RPT)])


def _make_agg(D):
    @functools.partial(
        pl.kernel,
        out_type=jax.ShapeDtypeStruct((NC, NP, D), jnp.float32),
        mesh=_mesh,
        scratch_types=[
            pltpu.VMEM((NCHUNK, K), jnp.int32),
            pltpu.VMEM((NCHUNK, K), jnp.int32),
            pltpu.VMEM((1, KT), jnp.int32),
            pltpu.VMEM((1, KT), jnp.int32),
            pltpu.VMEM((K, D), jnp.float32),
            pltpu.VMEM_SHARED((NP, D), jnp.float32),
        ],
    )
    def agg(g_hbm, src_hbm, dst_hbm, srct_hbm, dstt_hbm, zeros_hbm, out_hbm,
            sidx, didx, sidxt, didxt, rows, acc):
        c = lax.axis_index("c")
        s = lax.axis_index("s")
        wid = _worker(c, s)
        pltpu.sync_copy(src_hbm.at[wid], sidx)
        pltpu.sync_copy(dst_hbm.at[wid], didx)
        pltpu.sync_copy(srct_hbm.at[wid], sidxt)
        pltpu.sync_copy(dstt_hbm.at[wid], didxt)
        pltpu.sync_copy(zeros_hbm.at[pl.ds(s * RPT, RPT)],
                        acc.at[pl.ds(s * RPT, RPT)])
        plsc.subcore_barrier()

        def chunk(i, carry):
            pltpu.sync_copy(g_hbm.at[sidx.at[i]], rows)
            pltpu.sync_copy(rows, acc.at[didx.at[i]], add=True)
            return carry

        lax.fori_loop(0, NCHUNK, chunk, 0)
        pltpu.sync_copy(g_hbm.at[sidxt.at[0]], rows.at[pl.ds(0, KT)])
        pltpu.sync_copy(rows.at[pl.ds(0, KT)], acc.at[didxt.at[0]], add=True)
        plsc.subcore_barrier()
        pltpu.sync_copy(acc.at[pl.ds(s * RPT, RPT)],
                        out_hbm.at[c, pl.ds(s * RPT, RPT)])

    return agg


_agg_hid = _make_agg(D_HID)


_BLK = 1000


def _dinv(d0_ref, d1_ref):
    deg = d0_ref[:, 0:1] + d1_ref[:, 0:1] + 1.0
    return lax.rsqrt(deg)


def _g1_body(d0_ref, d1_ref, x_ref, w_ref, o_ref):
    h = jnp.dot(x_ref[...], w_ref[...], preferred_element_type=jnp.float32)
    o_ref[...] = h * _dinv(d0_ref, d1_ref)


def _mid_body(d0_ref, d1_ref, a0_ref, a1_ref, g1_ref, b1_ref, o_ref):
    dinv = _dinv(d0_ref, d1_ref)
    pre = (a0_ref[...] + a1_ref[...] + g1_ref[...]) * dinv + b1_ref[...]
    o_ref[...] = jnp.maximum(pre, 0.0) * dinv


def _out_body(d0_ref, d1_ref, a0_ref, a1_ref, s1_ref, w2_ref, b2_ref, o_ref):
    dinv = _dinv(d0_ref, d1_ref)
    pre = (a0_ref[...] + a1_ref[...] + s1_ref[...]) * dinv
    h = jnp.dot(pre, w2_ref[...], preferred_element_type=jnp.float32)
    o_ref[...] = h + b2_ref[...]


def _row_spec(d):
    return pl.BlockSpec((_BLK, d), lambda i: (i, 0))


def _full_spec(r, d):
    return pl.BlockSpec((r, d), lambda i: (0, 0))


_g1_call = pl.pallas_call(
    _g1_body,
    grid=(N // _BLK,),
    in_specs=[_row_spec(DEG_W), _row_spec(DEG_W), _row_spec(D_IN),
              _full_spec(D_IN, D_HID)],
    out_specs=_row_spec(D_HID),
    out_shape=jax.ShapeDtypeStruct((N, D_HID), jnp.float32),
)

_mid_call = pl.pallas_call(
    _mid_body,
    grid=(N // _BLK,),
    in_specs=[_row_spec(DEG_W), _row_spec(DEG_W), _row_spec(D_HID),
              _row_spec(D_HID), _row_spec(D_HID), _full_spec(1, D_HID)],
    out_specs=_row_spec(D_HID),
    out_shape=jax.ShapeDtypeStruct((N, D_HID), jnp.float32),
)

_out_call = pl.pallas_call(
    _out_body,
    grid=(N // _BLK,),
    in_specs=[_row_spec(DEG_W), _row_spec(DEG_W), _row_spec(D_HID),
              _row_spec(D_HID), _row_spec(D_HID), _full_spec(D_HID, D_OUT),
              _full_spec(1, D_OUT)],
    out_specs=_row_spec(D_OUT),
    out_shape=jax.ShapeDtypeStruct((N, D_OUT), jnp.float32),
)


@jax.jit
def kernel(x, edge_index, W1, b1, W2, b2):
    src_all = edge_index[0].reshape(NW, EPW)
    dst_all = edge_index[1].reshape(NW, EPW)
    src = src_all[:, :NCHUNK * K].reshape(NW, NCHUNK, K)
    dst = dst_all[:, :NCHUNK * K].reshape(NW, NCHUNK, K)
    srct = src_all[:, NCHUNK * K:].reshape(NW, 1, KT)
    dstt = dst_all[:, NCHUNK * K:].reshape(NW, 1, KT)
    ones_deg = jnp.ones((K, DEG_W), jnp.float32)
    zeros_deg = jnp.zeros((NP, DEG_W), jnp.float32)
    zeros_hid = jnp.zeros((NP, D_HID), jnp.float32)

    degacc = _deg_kernel(dst, dstt, ones_deg, zeros_deg)
    d0, d1 = degacc[0], degacc[1]

    g1 = _g1_call(d0, d1, x, W1)
    agg1 = _agg_hid(g1, src, dst, srct, dstt, zeros_hid)
    s1 = _mid_call(d0, d1, agg1[0], agg1[1], g1,
                   b1.reshape(1, D_HID))
    agg2 = _agg_hid(s1, src, dst, srct, dstt, zeros_hid)
    out = _out_call(d0, d1, agg2[0], agg2[1], s1, W2,
                    b2.reshape(1, D_OUT))
    return out


# speedup vs baseline: 18.9028x; 1.0009x over previous
import functools

import jax
import jax.numpy as jnp
from jax import lax
from jax.experimental import pallas as pl
from jax.experimental.pallas import tpu as pltpu
from jax.experimental.pallas import tpu_sc as plsc

N = 10000
E = 320000
D_IN = 128
D_HID = 128
D_OUT = 64

NC = 2
NS = 16
NW = NC * NS
EPW = E // NW
K = 128
NCHUNK = EPW // K
KT = EPW - NCHUNK * K
NP = 10112
RPT = NP // NS
DEG_W = 128

_mesh = plsc.VectorSubcoreMesh(core_axis_name="c", subcore_axis_name="s")


def _worker(c, s):
    return s * NC + c


@functools.partial(
    pl.kernel,
    out_type=jax.ShapeDtypeStruct((NC, NP, DEG_W), jnp.float32),
    mesh=_mesh,
    scratch_types=[
        pltpu.VMEM((NCHUNK, K), jnp.int32),
        pltpu.VMEM((1, KT), jnp.int32),
        pltpu.VMEM((K, DEG_W), jnp.float32),
        pltpu.VMEM_SHARED((NP, DEG_W), jnp.float32),
        pltpu.SemaphoreType.DMA,
    ],
)
def _deg_kernel(dst_hbm, dstt_hbm, ones_hbm, zeros_hbm, out_hbm,
                didx, didxt, ones_v, acc, sem):
    c = lax.axis_index("c")
    s = lax.axis_index("s")
    wid = _worker(c, s)
    pltpu.sync_copy(ones_hbm, ones_v)
    pltpu.sync_copy(dst_hbm.at[wid], didx)
    pltpu.sync_copy(dstt_hbm.at[wid], didxt)
    pltpu.sync_copy(zeros_hbm.at[pl.ds(s * RPT, RPT)],
                    acc.at[pl.ds(s * RPT, RPT)])
    plsc.subcore_barrier()

    def chunk(i, carry):
        pltpu.async_copy(ones_v, acc.at[didx.at[i]], sem, add=True)
        return carry

    lax.fori_loop(0, NCHUNK, chunk, 0)
    pltpu.async_copy(ones_v.at[pl.ds(0, KT)], acc.at[didxt.at[0]], sem,
                     add=True)

    def drain(i, carry):
        pltpu.make_async_copy(ones_v, acc.at[didx.at[i]], sem).wait()
        return carry

    lax.fori_loop(0, NCHUNK, drain, 0)
    pltpu.make_async_copy(ones_v.at[pl.ds(0, KT)], acc.at[didxt.at[0]],
                          sem).wait()
    plsc.subcore_barrier()
    pltpu.sync_copy(acc.at[pl.ds(s * RPT, RPT)],
                    out_hbm.at[c, pl.ds(s * RPT, RPT)])


def _make_agg(D):
    @functools.partial(
        pl.kernel,
        out_type=jax.ShapeDtypeStruct((NC, NP, D), jnp.float32),
        mesh=_mesh,
        scratch_types=[
            pltpu.VMEM((NCHUNK, K), jnp.int32),
            pltpu.VMEM((NCHUNK, K), jnp.int32),
            pltpu.VMEM((1, KT), jnp.int32),
            pltpu.VMEM((1, KT), jnp.int32),
            pltpu.VMEM((1, K, D), jnp.float32),
            pltpu.VMEM_SHARED((NP, D), jnp.float32),
        ],
    )
    def agg(g_hbm, src_hbm, dst_hbm, srct_hbm, dstt_hbm, zeros_hbm, out_hbm,
            sidx, didx, sidxt, didxt, rows, acc):
        c = lax.axis_index("c")
        s = lax.axis_index("s")
        wid = _worker(c, s)
        pltpu.sync_copy(src_hbm.at[wid], sidx)
        pltpu.sync_copy(dst_hbm.at[wid], didx)
        pltpu.sync_copy(srct_hbm.at[wid], sidxt)
        pltpu.sync_copy(dstt_hbm.at[wid], didxt)
        pltpu.sync_copy(zeros_hbm.at[pl.ds(s * RPT, RPT)],
                        acc.at[pl.ds(s * RPT, RPT)])
        plsc.subcore_barrier()

        def chunk(i, carry):
            pltpu.sync_copy(g_hbm.at[sidx.at[i]], rows.at[0])
            pltpu.sync_copy(rows.at[0], acc.at[didx.at[i]], add=True)
            return carry

        lax.fori_loop(0, NCHUNK, chunk, 0)
        pltpu.sync_copy(g_hbm.at[sidxt.at[0]], rows.at[0, pl.ds(0, KT)])
        pltpu.sync_copy(rows.at[0, pl.ds(0, KT)], acc.at[didxt.at[0]],
                        add=True)
        plsc.subcore_barrier()
        pltpu.sync_copy(acc.at[pl.ds(s * RPT, RPT)],
                        out_hbm.at[c, pl.ds(s * RPT, RPT)])

    return agg


_agg_hid = _make_agg(D_HID)


_BLK = 1000


def _dinv(d0_ref, d1_ref):
    deg = d0_ref[:, 0:1] + d1_ref[:, 0:1] + 1.0
    return lax.rsqrt(deg)


def _g1_body(d0_ref, d1_ref, x_ref, w_ref, o_ref):
    h = jnp.dot(x_ref[...], w_ref[...], preferred_element_type=jnp.float32)
    o_ref[...] = h * _dinv(d0_ref, d1_ref)


def _mid_body(d0_ref, d1_ref, a0_ref, a1_ref, g1_ref, b1_ref, o_ref):
    dinv = _dinv(d0_ref, d1_ref)
    pre = (a0_ref[...] + a1_ref[...] + g1_ref[...]) * dinv + b1_ref[...]
    o_ref[...] = jnp.maximum(pre, 0.0) * dinv


def _out_body(d0_ref, d1_ref, a0_ref, a1_ref, s1_ref, w2_ref, b2_ref, o_ref):
    dinv = _dinv(d0_ref, d1_ref)
    pre = (a0_ref[...] + a1_ref[...] + s1_ref[...]) * dinv
    h = jnp.dot(pre, w2_ref[...], preferred_element_type=jnp.float32)
    o_ref[...] = h + b2_ref[...]


def _row_spec(d):
    return pl.BlockSpec((_BLK, d), lambda i: (i, 0))


def _full_spec(r, d):
    return pl.BlockSpec((r, d), lambda i: (0, 0))


_g1_call = pl.pallas_call(
    _g1_body,
    grid=(N // _BLK,),
    in_specs=[_row_spec(DEG_W), _row_spec(DEG_W), _row_spec(D_IN),
              _full_spec(D_IN, D_HID)],
    out_specs=_row_spec(D_HID),
    out_shape=jax.ShapeDtypeStruct((N, D_HID), jnp.float32),
)

_mid_call = pl.pallas_call(
    _mid_body,
    grid=(N // _BLK,),
    in_specs=[_row_spec(DEG_W), _row_spec(DEG_W), _row_spec(D_HID),
              _row_spec(D_HID), _row_spec(D_HID), _full_spec(1, D_HID)],
    out_specs=_row_spec(D_HID),
    out_shape=jax.ShapeDtypeStruct((N, D_HID), jnp.float32),
)

_out_call = pl.pallas_call(
    _out_body,
    grid=(N // _BLK,),
    in_specs=[_row_spec(DEG_W), _row_spec(DEG_W), _row_spec(D_HID),
              _row_spec(D_HID), _row_spec(D_HID), _full_spec(D_HID, D_OUT),
              _full_spec(1, D_OUT)],
    out_specs=_row_spec(D_OUT),
    out_shape=jax.ShapeDtypeStruct((N, D_OUT), jnp.float32),
)


@jax.jit
def kernel(x, edge_index, W1, b1, W2, b2):
    src_all = edge_index[0].reshape(NW, EPW)
    dst_all = edge_index[1].reshape(NW, EPW)
    src = src_all[:, :NCHUNK * K].reshape(NW, NCHUNK, K)
    dst = dst_all[:, :NCHUNK * K].reshape(NW, NCHUNK, K)
    srct = src_all[:, NCHUNK * K:].reshape(NW, 1, KT)
    dstt = dst_all[:, NCHUNK * K:].reshape(NW, 1, KT)
    ones_deg = jnp.ones((K, DEG_W), jnp.float32)
    zeros_deg = jnp.zeros((NP, DEG_W), jnp.float32)
    zeros_hid = jnp.zeros((NP, D_HID), jnp.float32)

    degacc = _deg_kernel(dst, dstt, ones_deg, zeros_deg)
    d0, d1 = degacc[0], degacc[1]

    g1 = _g1_call(d0, d1, x, W1)
    agg1 = _agg_hid(g1, src, dst, srct, dstt, zeros_hid)
    s1 = _mid_call(d0, d1, agg1[0], agg1[1], g1,
                   b1.reshape(1, D_HID))
    agg2 = _agg_hid(s1, src, dst, srct, dstt, zeros_hid)
    out = _out_call(d0, d1, agg2[0], agg2[1], s1, W2,
                    b2.reshape(1, D_OUT))
    return out
